# baseline (speedup 1.0000x reference)
import jax
import jax.numpy as jnp
from jax.experimental import pallas as pl
from jax.experimental.pallas import tpu as pltpu

N = 10000
K_NEIGH = 32
D_FEAT = 128
IN_DIM = 2 * D_FEAT
OUT_DIM = 128

BLOCK_N = 400
HALF_N = BLOCK_N // 2


def _agg_body(feat_ref, emb_a_ref, emb_b_ref, w_ref, b_ref, out_ref):
    m_a = jnp.mean(emb_a_ref[...], axis=1)
    m_b = jnp.mean(emb_b_ref[...], axis=1)
    m = jnp.concatenate([m_a, m_b], axis=0)
    x = jnp.concatenate([feat_ref[...], m], axis=1)
    y = jnp.dot(x, w_ref[...], preferred_element_type=jnp.float32)
    out_ref[...] = jnp.maximum(y + b_ref[...], 0.0)


def kernel(features, embedding_look_up, kernel, bias_weights):
    bias2d = bias_weights.reshape(1, OUT_DIM)
    return pl.pallas_call(
        _agg_body,
        grid=(N // BLOCK_N,),
        in_specs=[
            pl.BlockSpec((BLOCK_N, D_FEAT), lambda i: (i, 0)),
            pl.BlockSpec((HALF_N, K_NEIGH, D_FEAT), lambda i: (2 * i, 0, 0)),
            pl.BlockSpec((HALF_N, K_NEIGH, D_FEAT),
                         lambda i: (2 * i + 1, 0, 0)),
            pl.BlockSpec((IN_DIM, OUT_DIM), lambda i: (0, 0)),
            pl.BlockSpec((1, OUT_DIM), lambda i: (0, 0)),
        ],
        out_specs=pl.BlockSpec((BLOCK_N, OUT_DIM), lambda i: (i, 0)),
        out_shape=jax.ShapeDtypeStruct((N, OUT_DIM), jnp.float32),
        compiler_params=pltpu.CompilerParams(
            dimension_semantics=("parallel",),
        ),
    )(features, embedding_look_up, embedding_look_up, kernel, bias2d)


# speedup vs baseline: 1.0105x; 1.0105x over previous
import jax
import jax.numpy as jnp
from jax.experimental import pallas as pl
from jax.experimental.pallas import tpu as pltpu

N = 10000
K_NEIGH = 32
D_FEAT = 128
IN_DIM = 2 * D_FEAT
OUT_DIM = 128

BLOCK_N = 400


def _agg_body(feat_ref, emb_ref, w_ref, b_ref, out_ref):
    emb = emb_ref[...]
    m = jnp.mean(emb, axis=1)
    x = jnp.concatenate([feat_ref[...], m], axis=1)
    y = jnp.dot(x, w_ref[...], preferred_element_type=jnp.float32)
    out_ref[...] = jnp.maximum(y + b_ref[...], 0.0)


def kernel(features, embedding_look_up, kernel, bias_weights):
    bias2d = bias_weights.reshape(1, OUT_DIM)
    return pl.pallas_call(
        _agg_body,
        grid=(N // BLOCK_N,),
        in_specs=[
            pl.BlockSpec((BLOCK_N, D_FEAT), lambda i: (i, 0)),
            pl.BlockSpec((BLOCK_N, K_NEIGH, D_FEAT), lambda i: (i, 0, 0)),
            pl.BlockSpec((IN_DIM, OUT_DIM), lambda i: (0, 0)),
            pl.BlockSpec((1, OUT_DIM), lambda i: (0, 0)),
        ],
        out_specs=pl.BlockSpec((BLOCK_N, OUT_DIM), lambda i: (i, 0)),
        out_shape=jax.ShapeDtypeStruct((N, OUT_DIM), jnp.float32),
        compiler_params=pltpu.CompilerParams(
            dimension_semantics=("arbitrary",),
        ),
    )(features, embedding_look_up, kernel, bias2d)
